# baseline (speedup 1.0000x reference)
import functools

import numpy as np

import jax
import jax.numpy as jnp
from jax import lax
from jax.experimental import pallas as pl
from jax.experimental.pallas import tpu as pltpu
from jax.experimental.pallas import tpu_sc as plsc

N = 10000
E = 320000
R = 8
IN = 128
HID = 64
OUT = 40

TW = 128
WID_SLICES = 3
NPAD = 10240
NW = 32
CH = 128
G = 80
EPT = G * CH
EPAD = NW * EPT
ROWS_PT = NPAD // 16
CG = 16
NCK = G // CG


def _dense_body(x_ref, w_ref, a_ref, pm_ref, tab_ref, ss_ref, sd_ref):
    xa = jnp.dot(x_ref[...], w_ref[0], preferred_element_type=jnp.float32)
    a = a_ref[pl.program_id(0)]
    ssrc = jnp.sum(xa * a[:HID][None, :], axis=1)
    sdst = jnp.sum(xa * a[HID:][None, :], axis=1)
    nrows = xa.shape[0]
    padded = jnp.concatenate(
        [xa,
         jnp.ones((nrows, 1), jnp.float32),
         jnp.zeros((nrows, TW - HID - 1), jnp.float32)], axis=1)
    tab = jnp.dot(padded, pm_ref[...], preferred_element_type=jnp.float32)
    tab_ref[0] = tab.astype(jnp.bfloat16)
    ss_ref[0, 0] = ssrc
    sd_ref[0, 0] = sdst


def _dense(xh, W, att, pm):
    D = xh.shape[1]
    return pl.pallas_call(
        _dense_body,
        grid=(R,),
        in_specs=[pl.BlockSpec((N, D), lambda r: (0, 0)),
                  pl.BlockSpec((1, D, HID), lambda r: (r, 0, 0)),
                  pl.BlockSpec((R, 2 * HID), lambda r: (0, 0)),
                  pl.BlockSpec((TW, TW), lambda r: (0, 0))],
        out_specs=[pl.BlockSpec((1, N, TW), lambda r: (r, 0, 0)),
                   pl.BlockSpec((1, 1, N), lambda r: (r, 0, 0)),
                   pl.BlockSpec((1, 1, N), lambda r: (r, 0, 0))],
        out_shape=[jax.ShapeDtypeStruct((R, N, TW), jnp.bfloat16),
                   jax.ShapeDtypeStruct((R, 1, N), jnp.float32),
                   jax.ShapeDtypeStruct((R, 1, N), jnp.float32)],
    )(xh, W, att, pm)


_mesh = plsc.VectorSubcoreMesh(core_axis_name="c", subcore_axis_name="s")


@functools.partial(
    pl.kernel, mesh=_mesh,
    compiler_params=pltpu.CompilerParams(needs_layout_passes=False,
                                         use_tc_tiling_on_sc=False),
    out_type=jax.ShapeDtypeStruct((2, NPAD, TW), jnp.float32),
    scratch_types=[
        pltpu.VMEM((CG, CH), jnp.int32),
        pltpu.VMEM((CG, CH), jnp.int32),
        pltpu.VMEM((CG, CH), jnp.int32),
        pltpu.VMEM((CH, TW // 2), jnp.int32),
        pltpu.VMEM((CH, TW // 2), jnp.int32),
        pltpu.VMEM((CH, TW), jnp.float32),
        pltpu.VMEM((CH,), jnp.float32),
        pltpu.VMEM((CH,), jnp.float32),
        pltpu.VMEM((CH,), jnp.float32),
        pltpu.VMEM((CH,), jnp.float32),
        pltpu.VMEM_SHARED((NPAD, TW), jnp.float32),
        pltpu.SemaphoreType.DMA,
        pltpu.SemaphoreType.DMA,
    ],
)
def _edge_kernel(tab_hbm, ssrc_hbm, sdst_hbm, et_hbm, src_hbm, dst_hbm,
                 out_hbm, etv, srcv, dstv, rows_a, rows_b, sbuf, ssv_a,
                 ssv_b, sdv_a, sdv_b, acc, sem_a, sem_b):
    cid = lax.axis_index("c")
    tid = lax.axis_index("s")
    wid = cid * 16 + tid

    zf = jnp.zeros((16,), jnp.float32)

    def zbody(i, _):
        for k in range(TW // 16):
            sbuf[i, pl.ds(k * 16, 16)] = zf
        return 0

    lax.fori_loop(0, CH, zbody, 0)
    for m in range(ROWS_PT // CH):
        pltpu.sync_copy(sbuf, acc.at[pl.ds(tid * ROWS_PT + m * CH, CH)])
    plsc.subcore_barrier()

    def _issue(g, buf, ssv, sdv, sem):
        pltpu.async_copy(ssrc_hbm.at[srcv.at[g]], ssv, sem)
        pltpu.async_copy(sdst_hbm.at[etv.at[g]], sdv, sem)
        pltpu.async_copy(tab_hbm.at[srcv.at[g]], buf, sem)

    def _wait(g, buf, ssv, sdv, sem):
        pltpu.make_async_copy(ssrc_hbm.at[srcv.at[g]], ssv, sem).wait()
        pltpu.make_async_copy(sdst_hbm.at[etv.at[g]], sdv, sem).wait()
        pltpu.make_async_copy(tab_hbm.at[srcv.at[g]], buf, sem).wait()

    hi_mask = jnp.full((16,), 0xFFFF0000, jnp.uint32)

    def _process(g, buf, ssv, sdv):
        def jbody(j, _):
            base = j * 16
            sl = pl.ds(base, 16)
            z = ssv[sl] + sdv[sl]
            z = jnp.maximum(z, 0.2 * z)
            w16 = jnp.exp(z)
            for e in range(16):
                ws = jnp.full((16,), w16[e])
                for k in range(WID_SLICES):
                    u = plsc.bitcast(buf[base + e, pl.ds(16 * k, 16)],
                                     jnp.uint32)
                    lo = plsc.bitcast(u << 16, jnp.float32)
                    hi = plsc.bitcast(u & hi_mask, jnp.float32)
                    sbuf[base + e, pl.ds(32 * k, 16)] = lo * ws
                    sbuf[base + e, pl.ds(32 * k + 16, 16)] = hi * ws
            return 0

        lax.fori_loop(0, CH // 16, jbody, 0)
        pltpu.sync_copy(sbuf, acc.at[dstv.at[g]], add=True)

    for c in range(NCK):
        pltpu.sync_copy(et_hbm.at[wid, pl.ds(c * CG, CG)], etv)
        pltpu.sync_copy(src_hbm.at[wid, pl.ds(c * CG, CG)], srcv)
        pltpu.sync_copy(dst_hbm.at[wid, pl.ds(c * CG, CG)], dstv)

        def ibody(g, _):
            for j in range(CH // 16):
                sl = pl.ds(j * 16, 16)
                e16 = etv[g, sl]
                srcv[g, sl] = e16 * N + srcv[g, sl]
                etv[g, sl] = e16 * N + dstv[g, sl]
            return 0

        lax.fori_loop(0, CG, ibody, 0)
        _issue(0, rows_a, ssv_a, sdv_a, sem_a)

        def gbody(i, _):
            g0 = 2 * i
            g1 = g0 + 1
            _issue(g1, rows_b, ssv_b, sdv_b, sem_b)
            _wait(g0, rows_a, ssv_a, sdv_a, sem_a)
            _process(g0, rows_a, ssv_a, sdv_a)

            @pl.when(g1 + 1 < CG)
            def _():
                _issue(g1 + 1, rows_a, ssv_a, sdv_a, sem_a)

            _wait(g1, rows_b, ssv_b, sdv_b, sem_b)
            _process(g1, rows_b, ssv_b, sdv_b)
            return 0

        lax.fori_loop(0, CG // 2, gbody, 0)
    plsc.subcore_barrier()
    pltpu.sync_copy(acc.at[pl.ds(tid * ROWS_PT, ROWS_PT)],
                    out_hbm.at[cid, pl.ds(tid * ROWS_PT, ROWS_PT)])


def _combine_body(acc_ref, h_ref):
    f = acc_ref[0, :N, :HID] + acc_ref[1, :N, :HID]
    d = acc_ref[0, :N, HID:HID + 1] + acc_ref[1, :N, HID:HID + 1]
    h_ref[...] = jnp.maximum(f / (d + 1e-16), 0.0)


def _combine(acc):
    return pl.pallas_call(
        _combine_body,
        out_shape=jax.ShapeDtypeStruct((N, HID), jnp.float32),
    )(acc)


def _final_body(acc_ref, lw_ref, lb_ref, o_ref):
    f = acc_ref[0, :N, :HID] + acc_ref[1, :N, :HID]
    d = acc_ref[0, :N, HID:HID + 1] + acc_ref[1, :N, HID:HID + 1]
    h = jnp.maximum(f / (d + 1e-16), 0.0)
    logits = jnp.dot(h, lw_ref[...], preferred_element_type=jnp.float32)
    logits = logits + lb_ref[...]
    m = jnp.max(logits, axis=1, keepdims=True)
    ex = jnp.exp(logits - m)
    o_ref[...] = logits - m - jnp.log(jnp.sum(ex, axis=1, keepdims=True))


def _final(acc, lin_W, lin_b):
    return pl.pallas_call(
        _final_body,
        out_shape=jax.ShapeDtypeStruct((N, OUT), jnp.float32),
    )(acc, lin_W, lin_b)


def _perm_matrix():
    pm = np.zeros((TW, TW), np.float32)
    for k in range(TW // 32):
        for j in range(16):
            pm[32 * k + j, 32 * k + 2 * j] = 1.0
            pm[32 * k + 16 + j, 32 * k + 2 * j + 1] = 1.0
    return jnp.asarray(pm)


def kernel(x, edge_index, edge_type, tensor_slice, W1, att1, W2, att2,
           lin_W, lin_b):
    pad = EPAD - E
    et_p = jnp.concatenate(
        [edge_type, jnp.zeros((pad,), jnp.int32)]).reshape(NW, G, CH)
    src_p = jnp.concatenate(
        [edge_index[0], jnp.zeros((pad,), jnp.int32)]).reshape(NW, G, CH)
    dst_p = jnp.concatenate(
        [edge_index[1],
         jnp.full((pad,), NPAD - 1, jnp.int32)]).reshape(NW, G, CH)

    def _pack_i32(tab_bf):
        return jax.lax.bitcast_convert_type(
            tab_bf.reshape(R * N, TW // 2, 2), jnp.int32)

    pm = _perm_matrix()
    tab1, ssrc1, sdst1 = _dense(x, W1, att1, pm)
    acc1 = _edge_kernel(_pack_i32(tab1), ssrc1.reshape(R * N),
                        sdst1.reshape(R * N), et_p, src_p, dst_p)
    h = _combine(acc1)
    tab2, ssrc2, sdst2 = _dense(h, W2, att2, pm)
    acc2 = _edge_kernel(_pack_i32(tab2), ssrc2.reshape(R * N),
                        sdst2.reshape(R * N), et_p, src_p, dst_p)
    return _final(acc2, lin_W, lin_b.reshape(1, OUT))


# speedup vs baseline: 1.1334x; 1.1334x over previous
import functools

import jax
import jax.numpy as jnp
from jax import lax
from jax.experimental import pallas as pl
from jax.experimental.pallas import tpu as pltpu
from jax.experimental.pallas import tpu_sc as plsc

N = 10000
E = 320000
R = 8
IN = 128
HID = 64
OUT = 40

TW = 128
SCALE_SLICES = 5
NPAD = 10240
NW = 32
CH = 64
G = 160
EPT = G * CH
EPAD = NW * EPT
ROWS_PT = NPAD // 16
CG = 32
NCK = G // CG
NBUF = 4


def _dense_body(x_ref, w_ref, a_ref, tab_ref, ss_ref, sd_ref):
    xa = jnp.dot(x_ref[...], w_ref[0], preferred_element_type=jnp.float32)
    a = a_ref[pl.program_id(0)]
    ssrc = jnp.sum(xa * a[:HID][None, :], axis=1)
    sdst = jnp.sum(xa * a[HID:][None, :], axis=1)
    nrows = xa.shape[0]
    tab = jnp.concatenate(
        [xa,
         jnp.ones((nrows, 1), jnp.float32),
         jnp.zeros((nrows, TW - HID - 1), jnp.float32)], axis=1)
    tab_ref[0] = tab
    ss_ref[0, 0] = ssrc
    sd_ref[0, 0] = sdst


def _dense(xh, W, att):
    D = xh.shape[1]
    return pl.pallas_call(
        _dense_body,
        grid=(R,),
        in_specs=[pl.BlockSpec((N, D), lambda r: (0, 0)),
                  pl.BlockSpec((1, D, HID), lambda r: (r, 0, 0)),
                  pl.BlockSpec((R, 2 * HID), lambda r: (0, 0))],
        out_specs=[pl.BlockSpec((1, N, TW), lambda r: (r, 0, 0)),
                   pl.BlockSpec((1, 1, N), lambda r: (r, 0, 0)),
                   pl.BlockSpec((1, 1, N), lambda r: (r, 0, 0))],
        out_shape=[jax.ShapeDtypeStruct((R, N, TW), jnp.float32),
                   jax.ShapeDtypeStruct((R, 1, N), jnp.float32),
                   jax.ShapeDtypeStruct((R, 1, N), jnp.float32)],
    )(xh, W, att)


_mesh = plsc.VectorSubcoreMesh(core_axis_name="c", subcore_axis_name="s")


@functools.partial(
    pl.kernel, mesh=_mesh,
    out_type=jax.ShapeDtypeStruct((2, NPAD, TW), jnp.float32),
    scratch_types=[
        pltpu.VMEM((CG, CH), jnp.int32),
        pltpu.VMEM((CG, CH), jnp.int32),
        pltpu.VMEM((CG, CH), jnp.int32),
        *([pltpu.VMEM((CH, TW), jnp.float32)] * NBUF),
        *([pltpu.VMEM((CH,), jnp.float32)] * NBUF),
        *([pltpu.VMEM((CH,), jnp.float32)] * NBUF),
        pltpu.VMEM_SHARED((NPAD, TW), jnp.float32),
        *([pltpu.SemaphoreType.DMA] * NBUF),
    ],
)
def _edge_kernel(tab_hbm, ssrc_hbm, sdst_hbm, et_hbm, src_hbm, dst_hbm,
                 out_hbm, etv, srcv, dstv, *rest):
    rows_bufs = rest[0:NBUF]
    ssv_bufs = rest[NBUF:2 * NBUF]
    sdv_bufs = rest[2 * NBUF:3 * NBUF]
    acc = rest[3 * NBUF]
    sems = rest[3 * NBUF + 1:3 * NBUF + 1 + NBUF]
    cid = lax.axis_index("c")
    tid = lax.axis_index("s")
    wid = cid * 16 + tid

    zf = jnp.zeros((16,), jnp.float32)

    def zbody(i, _):
        for k in range(TW // 16):
            rows_bufs[0][i, pl.ds(k * 16, 16)] = zf
        return 0

    lax.fori_loop(0, CH, zbody, 0)
    for m in range(ROWS_PT // CH):
        pltpu.sync_copy(rows_bufs[0],
                        acc.at[pl.ds(tid * ROWS_PT + m * CH, CH)])
    plsc.subcore_barrier()

    def _issue(g, buf, ssv, sdv, sem):
        pltpu.async_copy(ssrc_hbm.at[srcv.at[g]], ssv, sem)
        pltpu.async_copy(sdst_hbm.at[etv.at[g]], sdv, sem)
        pltpu.async_copy(tab_hbm.at[srcv.at[g]], buf, sem)

    def _wait(g, buf, ssv, sdv, sem):
        pltpu.make_async_copy(ssrc_hbm.at[srcv.at[g]], ssv, sem).wait()
        pltpu.make_async_copy(sdst_hbm.at[etv.at[g]], sdv, sem).wait()
        pltpu.make_async_copy(tab_hbm.at[srcv.at[g]], buf, sem).wait()

    def _process(g, buf, ssv, sdv):
        def jbody(j, _):
            base = j * 16
            sl = pl.ds(base, 16)
            z = ssv[sl] + sdv[sl]
            z = jnp.maximum(z, 0.2 * z)
            w16 = jnp.exp(z)
            for e in range(16):
                ws = jnp.full((16,), w16[e])
                for k in range(SCALE_SLICES):
                    ksl = pl.ds(k * 16, 16)
                    buf[base + e, ksl] = buf[base + e, ksl] * ws
            return 0

        lax.fori_loop(0, CH // 16, jbody, 0)
        pltpu.sync_copy(buf, acc.at[dstv.at[g]], add=True)

    for c in range(NCK):
        pltpu.sync_copy(et_hbm.at[wid, pl.ds(c * CG, CG)], etv)
        pltpu.sync_copy(src_hbm.at[wid, pl.ds(c * CG, CG)], srcv)
        pltpu.sync_copy(dst_hbm.at[wid, pl.ds(c * CG, CG)], dstv)

        def ibody(g, _):
            for j in range(CH // 16):
                sl = pl.ds(j * 16, 16)
                e16 = etv[g, sl]
                srcv[g, sl] = e16 * N + srcv[g, sl]
                etv[g, sl] = e16 * N + dstv[g, sl]
            return 0

        lax.fori_loop(0, CG, ibody, 0)
        for b in range(NBUF - 1):
            _issue(b, rows_bufs[b], ssv_bufs[b], sdv_bufs[b], sems[b])

        def gbody(i, _):
            gq = NBUF * i
            for b in range(NBUF):
                g = gq + b
                nb = (b + NBUF - 1) % NBUF
                gn = g + NBUF - 1

                @pl.when(gn < CG)
                def _(nb=nb, gn=gn):
                    _issue(gn, rows_bufs[nb], ssv_bufs[nb], sdv_bufs[nb],
                           sems[nb])

                _wait(g, rows_bufs[b], ssv_bufs[b], sdv_bufs[b], sems[b])
                _process(g, rows_bufs[b], ssv_bufs[b], sdv_bufs[b])
            return 0

        lax.fori_loop(0, CG // NBUF, gbody, 0)
    plsc.subcore_barrier()
    pltpu.sync_copy(acc.at[pl.ds(tid * ROWS_PT, ROWS_PT)],
                    out_hbm.at[cid, pl.ds(tid * ROWS_PT, ROWS_PT)])


def _combine_body(acc_ref, h_ref):
    f = acc_ref[0, :N, :HID] + acc_ref[1, :N, :HID]
    d = acc_ref[0, :N, HID:HID + 1] + acc_ref[1, :N, HID:HID + 1]
    h_ref[...] = jnp.maximum(f / (d + 1e-16), 0.0)


def _combine(acc):
    return pl.pallas_call(
        _combine_body,
        out_shape=jax.ShapeDtypeStruct((N, HID), jnp.float32),
    )(acc)


def _final_body(acc_ref, lw_ref, lb_ref, o_ref):
    f = acc_ref[0, :N, :HID] + acc_ref[1, :N, :HID]
    d = acc_ref[0, :N, HID:HID + 1] + acc_ref[1, :N, HID:HID + 1]
    h = jnp.maximum(f / (d + 1e-16), 0.0)
    logits = jnp.dot(h, lw_ref[...], preferred_element_type=jnp.float32)
    logits = logits + lb_ref[...]
    m = jnp.max(logits, axis=1, keepdims=True)
    ex = jnp.exp(logits - m)
    o_ref[...] = logits - m - jnp.log(jnp.sum(ex, axis=1, keepdims=True))


def _final(acc, lin_W, lin_b):
    return pl.pallas_call(
        _final_body,
        out_shape=jax.ShapeDtypeStruct((N, OUT), jnp.float32),
    )(acc, lin_W, lin_b)


def kernel(x, edge_index, edge_type, tensor_slice, W1, att1, W2, att2,
           lin_W, lin_b):
    pad = EPAD - E
    et_p = jnp.concatenate(
        [edge_type, jnp.zeros((pad,), jnp.int32)]).reshape(NW, G, CH)
    src_p = jnp.concatenate(
        [edge_index[0], jnp.zeros((pad,), jnp.int32)]).reshape(NW, G, CH)
    dst_p = jnp.concatenate(
        [edge_index[1],
         jnp.full((pad,), NPAD - 1, jnp.int32)]).reshape(NW, G, CH)

    tab1, ssrc1, sdst1 = _dense(x, W1, att1)
    acc1 = _edge_kernel(tab1.reshape(R * N, TW), ssrc1.reshape(R * N),
                        sdst1.reshape(R * N), et_p, src_p, dst_p)
    h = _combine(acc1)
    tab2, ssrc2, sdst2 = _dense(h, W2, att2)
    acc2 = _edge_kernel(tab2.reshape(R * N, TW), ssrc2.reshape(R * N),
                        sdst2.reshape(R * N), et_p, src_p, dst_p)
    return _final(acc2, lin_W, lin_b.reshape(1, OUT))


# speedup vs baseline: 1.1739x; 1.0357x over previous
import functools

import jax
import jax.numpy as jnp
from jax import lax
from jax.experimental import pallas as pl
from jax.experimental.pallas import tpu as pltpu
from jax.experimental.pallas import tpu_sc as plsc

N = 10000
E = 320000
R = 8
IN = 128
HID = 64
OUT = 40

TW = 128
SCALE_SLICES = 5
NPAD = 10240
NW = 32
CH = 128
G = 80
EPT = G * CH
EPAD = NW * EPT
ROWS_PT = NPAD // 16
CG = 16
NCK = G // CG


def _dense_body(x_ref, w_ref, a_ref, tab_ref, ss_ref, sd_ref):
    xa = jnp.dot(x_ref[...], w_ref[0], preferred_element_type=jnp.float32)
    a = a_ref[pl.program_id(0)]
    ssrc = jnp.sum(xa * a[:HID][None, :], axis=1)
    sdst = jnp.sum(xa * a[HID:][None, :], axis=1)
    nrows = xa.shape[0]
    tab = jnp.concatenate(
        [xa,
         jnp.ones((nrows, 1), jnp.float32),
         jnp.zeros((nrows, TW - HID - 1), jnp.float32)], axis=1)
    tab_ref[0] = tab
    ss_ref[0, 0] = ssrc
    sd_ref[0, 0] = sdst


def _dense(xh, W, att):
    D = xh.shape[1]
    return pl.pallas_call(
        _dense_body,
        grid=(R,),
        in_specs=[pl.BlockSpec((N, D), lambda r: (0, 0)),
                  pl.BlockSpec((1, D, HID), lambda r: (r, 0, 0)),
                  pl.BlockSpec((R, 2 * HID), lambda r: (0, 0))],
        out_specs=[pl.BlockSpec((1, N, TW), lambda r: (r, 0, 0)),
                   pl.BlockSpec((1, 1, N), lambda r: (r, 0, 0)),
                   pl.BlockSpec((1, 1, N), lambda r: (r, 0, 0))],
        out_shape=[jax.ShapeDtypeStruct((R, N, TW), jnp.float32),
                   jax.ShapeDtypeStruct((R, 1, N), jnp.float32),
                   jax.ShapeDtypeStruct((R, 1, N), jnp.float32)],
    )(xh, W, att)


_mesh = plsc.VectorSubcoreMesh(core_axis_name="c", subcore_axis_name="s")


@functools.partial(
    pl.kernel, mesh=_mesh,
    out_type=jax.ShapeDtypeStruct((2, NPAD, TW), jnp.float32),
    scratch_types=[
        pltpu.VMEM((CG, CH), jnp.int32),
        pltpu.VMEM((CG, CH), jnp.int32),
        pltpu.VMEM((CG, CH), jnp.int32),
        pltpu.VMEM((CH, TW), jnp.float32),
        pltpu.VMEM((CH, TW), jnp.float32),
        pltpu.VMEM((CH,), jnp.float32),
        pltpu.VMEM((CH,), jnp.float32),
        pltpu.VMEM((CH,), jnp.float32),
        pltpu.VMEM((CH,), jnp.float32),
        pltpu.VMEM_SHARED((NPAD, TW), jnp.float32),
        pltpu.SemaphoreType.DMA,
        pltpu.SemaphoreType.DMA,
    ],
)
def _edge_kernel(tab_hbm, ssrc_hbm, sdst_hbm, et_hbm, src_hbm, dst_hbm,
                 out_hbm, etv, srcv, dstv, rows_a, rows_b, ssv_a, ssv_b,
                 sdv_a, sdv_b, acc, sem_a, sem_b):
    cid = lax.axis_index("c")
    tid = lax.axis_index("s")
    wid = cid * 16 + tid

    zf = jnp.zeros((16,), jnp.float32)

    def zbody(i, _):
        for k in range(TW // 16):
            rows_a[i, pl.ds(k * 16, 16)] = zf
        return 0

    lax.fori_loop(0, CH, zbody, 0)
    for m in range(ROWS_PT // CH):
        pltpu.sync_copy(rows_a, acc.at[pl.ds(tid * ROWS_PT + m * CH, CH)])
    plsc.subcore_barrier()

    def _issue(g, buf, ssv, sdv, sem):
        pltpu.async_copy(ssrc_hbm.at[srcv.at[g]], ssv, sem)
        pltpu.async_copy(sdst_hbm.at[etv.at[g]], sdv, sem)
        pltpu.async_copy(tab_hbm.at[srcv.at[g]], buf, sem)

    def _wait(g, buf, ssv, sdv, sem):
        pltpu.make_async_copy(ssrc_hbm.at[srcv.at[g]], ssv, sem).wait()
        pltpu.make_async_copy(sdst_hbm.at[etv.at[g]], sdv, sem).wait()
        pltpu.make_async_copy(tab_hbm.at[srcv.at[g]], buf, sem).wait()

    def _process(g, buf, ssv, sdv):
        def jbody(j, _):
            base = j * 16
            sl = pl.ds(base, 16)
            z = ssv[sl] + sdv[sl]
            z = jnp.maximum(z, 0.2 * z)
            w16 = jnp.exp(z)
            for e in range(16):
                ws = jnp.full((16,), w16[e])
                for k in range(SCALE_SLICES):
                    ksl = pl.ds(k * 16, 16)
                    buf[base + e, ksl] = buf[base + e, ksl] * ws
            return 0

        lax.fori_loop(0, CH // 16, jbody, 0)
        pltpu.sync_copy(buf, acc.at[dstv.at[g]], add=True)

    for c in range(NCK):
        pltpu.sync_copy(et_hbm.at[wid, pl.ds(c * CG, CG)], etv)
        pltpu.sync_copy(src_hbm.at[wid, pl.ds(c * CG, CG)], srcv)
        pltpu.sync_copy(dst_hbm.at[wid, pl.ds(c * CG, CG)], dstv)

        def ibody(g, _):
            for j in range(CH // 16):
                sl = pl.ds(j * 16, 16)
                e16 = etv[g, sl]
                srcv[g, sl] = e16 * N + srcv[g, sl]
                etv[g, sl] = e16 * N + dstv[g, sl]
            return 0

        lax.fori_loop(0, CG, ibody, 0)
        _issue(0, rows_a, ssv_a, sdv_a, sem_a)

        def gbody(i, _):
            g0 = 2 * i
            g1 = g0 + 1
            _issue(g1, rows_b, ssv_b, sdv_b, sem_b)
            _wait(g0, rows_a, ssv_a, sdv_a, sem_a)
            _process(g0, rows_a, ssv_a, sdv_a)

            @pl.when(g1 + 1 < CG)
            def _():
                _issue(g1 + 1, rows_a, ssv_a, sdv_a, sem_a)

            _wait(g1, rows_b, ssv_b, sdv_b, sem_b)
            _process(g1, rows_b, ssv_b, sdv_b)
            return 0

        lax.fori_loop(0, CG // 2, gbody, 0)
    plsc.subcore_barrier()
    pltpu.sync_copy(acc.at[pl.ds(tid * ROWS_PT, ROWS_PT)],
                    out_hbm.at[cid, pl.ds(tid * ROWS_PT, ROWS_PT)])


def _combine_body(acc_ref, h_ref):
    f = acc_ref[0, :N, :HID] + acc_ref[1, :N, :HID]
    d = acc_ref[0, :N, HID:HID + 1] + acc_ref[1, :N, HID:HID + 1]
    h_ref[...] = jnp.maximum(f / (d + 1e-16), 0.0)


def _combine(acc):
    return pl.pallas_call(
        _combine_body,
        out_shape=jax.ShapeDtypeStruct((N, HID), jnp.float32),
    )(acc)


def _final_body(acc_ref, lw_ref, lb_ref, o_ref):
    f = acc_ref[0, :N, :HID] + acc_ref[1, :N, :HID]
    d = acc_ref[0, :N, HID:HID + 1] + acc_ref[1, :N, HID:HID + 1]
    h = jnp.maximum(f / (d + 1e-16), 0.0)
    logits = jnp.dot(h, lw_ref[...], preferred_element_type=jnp.float32)
    logits = logits + lb_ref[...]
    m = jnp.max(logits, axis=1, keepdims=True)
    ex = jnp.exp(logits - m)
    o_ref[...] = logits - m - jnp.log(jnp.sum(ex, axis=1, keepdims=True))


def _final(acc, lin_W, lin_b):
    return pl.pallas_call(
        _final_body,
        out_shape=jax.ShapeDtypeStruct((N, OUT), jnp.float32),
    )(acc, lin_W, lin_b)


def kernel(x, edge_index, edge_type, tensor_slice, W1, att1, W2, att2,
           lin_W, lin_b):
    pad = EPAD - E
    et_p = jnp.concatenate(
        [edge_type, jnp.zeros((pad,), jnp.int32)]).reshape(NW, G, CH)
    src_p = jnp.concatenate(
        [edge_index[0], jnp.zeros((pad,), jnp.int32)]).reshape(NW, G, CH)
    dst_p = jnp.concatenate(
        [edge_index[1],
         jnp.full((pad,), NPAD - 1, jnp.int32)]).reshape(NW, G, CH)

    tab1, ssrc1, sdst1 = _dense(x, W1, att1)
    acc1 = _edge_kernel(tab1.reshape(R * N, TW), ssrc1.reshape(R * N),
                        sdst1.reshape(R * N), et_p, src_p, dst_p)
    h = _combine(acc1)
    tab2, ssrc2, sdst2 = _dense(h, W2, att2)
    acc2 = _edge_kernel(tab2.reshape(R * N, TW), ssrc2.reshape(R * N),
                        sdst2.reshape(R * N), et_p, src_p, dst_p)
    return _final(acc2, lin_W, lin_b.reshape(1, OUT))
